# baseline (speedup 1.0000x reference)
import functools

import jax
import jax.numpy as jnp
from jax import lax
from jax.experimental import pallas as pl
from jax.experimental.pallas import tpu as pltpu, tpu_sc as plsc

P = 0.5
ROWS = 64
COLS = 160000
NC, NS = 2, 16
NW = NC * NS
ROWS_PER_W = ROWS // NW
CHUNK = 32000
CHUNKS_PER_ROW = COLS // CHUNK
LANES = 16
VECS_PER_CHUNK = CHUNK // LANES


def _row_signs():
    mask = jax.random.uniform(jax.random.key(42), (ROWS,)) < P
    return jnp.where(mask, -1.0, 1.0).astype(jnp.float32)


def _sc_body(x_hbm, signs_hbm, out_hbm, buf, sv):
    wid = lax.axis_index("s") * NC + lax.axis_index("c")
    base = wid * (ROWS_PER_W * COLS)
    pltpu.sync_copy(signs_hbm.at[pl.ds(wid * (ROWS_PER_W * LANES),
                                       ROWS_PER_W * LANES)], sv)
    for c in range(ROWS_PER_W * CHUNKS_PER_ROW):
        row_in_w = c // CHUNKS_PER_ROW
        svec = sv[pl.ds(row_in_w * LANES, LANES)]
        off = base + c * CHUNK
        pltpu.sync_copy(x_hbm.at[pl.ds(off, CHUNK)], buf)

        def body(i, _):
            sl = pl.ds(i * LANES, LANES)
            buf[sl] = buf[sl] * svec
            return 0

        lax.fori_loop(0, VECS_PER_CHUNK, body, 0)
        pltpu.sync_copy(buf, out_hbm.at[pl.ds(off, CHUNK)])


def kernel(x):
    signs = _row_signs()
    signs_rep = jnp.broadcast_to(signs[:, None], (ROWS, LANES)).reshape(-1)
    x_flat = x.reshape(-1)
    k = pl.kernel(
        _sc_body,
        out_type=jax.ShapeDtypeStruct((ROWS * COLS,), jnp.float32),
        mesh=plsc.VectorSubcoreMesh(core_axis_name="c", subcore_axis_name="s"),
        scratch_types=[
            pltpu.VMEM((CHUNK,), jnp.float32),
            pltpu.VMEM((ROWS_PER_W * LANES,), jnp.float32),
        ],
    )
    out = k(x_flat, signs_rep)
    return out.reshape(x.shape)


# speedup vs baseline: 3.1386x; 3.1386x over previous
import jax
import jax.numpy as jnp
from jax import lax
from jax.experimental import pallas as pl
from jax.experimental.pallas import tpu as pltpu, tpu_sc as plsc

P = 0.5
ROWS = 64
COLS = 160000
NC, NS = 2, 16
NW = NC * NS
ROWS_PER_W = ROWS // NW
CHUNK = 40000
CHUNKS_PER_ROW = COLS // CHUNK
NCHUNK = ROWS_PER_W * CHUNKS_PER_ROW
LANES = 16
VECS_PER_CHUNK = CHUNK // LANES


def _row_signs():
    mask = jax.random.uniform(jax.random.key(42), (ROWS,)) < P
    return jnp.where(mask, -1.0, 1.0).astype(jnp.float32)


def _sc_body(x_hbm, signs_hbm, out_hbm, b0, b1, b2, sv,
             si0, si1, si2, so0, so1, so2):
    bufs = (b0, b1, b2)
    sin = (si0, si1, si2)
    sout = (so0, so1, so2)
    wid = lax.axis_index("s") * NC + lax.axis_index("c")
    base = wid * (ROWS_PER_W * COLS)
    pltpu.sync_copy(signs_hbm.at[pl.ds(wid * (ROWS_PER_W * LANES),
                                       ROWS_PER_W * LANES)], sv)

    def start_in(c):
        return pltpu.async_copy(
            x_hbm.at[pl.ds(base + c * CHUNK, CHUNK)], bufs[c % 3], sin[c % 3])

    def start_out(c):
        return pltpu.async_copy(
            bufs[c % 3], out_hbm.at[pl.ds(base + c * CHUNK, CHUNK)],
            sout[c % 3])

    in_h = {0: start_in(0), 1: start_in(1)}
    out_h = {}
    for c in range(NCHUNK):
        s = c % 3
        in_h.pop(c).wait()
        buf = bufs[s]
        row_in_w = c // CHUNKS_PER_ROW
        svec = sv[pl.ds(row_in_w * LANES, LANES)]

        @plsc.parallel_loop(0, CHUNK, LANES, unroll=8)
        def body(i):
            sl = pl.ds(i, LANES)
            buf[sl] = buf[sl] * svec
        out_h[c] = start_out(c)
        if c + 2 < NCHUNK:
            if c - 1 >= 0:
                out_h.pop(c - 1).wait()
            in_h[c + 2] = start_in(c + 2)
    for c in list(out_h):
        out_h.pop(c).wait()


def kernel(x):
    signs = _row_signs()
    signs_rep = jnp.broadcast_to(signs[:, None], (ROWS, LANES)).reshape(-1)
    x_flat = x.reshape(-1)
    k = pl.kernel(
        _sc_body,
        out_type=jax.ShapeDtypeStruct((ROWS * COLS,), jnp.float32),
        mesh=plsc.VectorSubcoreMesh(core_axis_name="c", subcore_axis_name="s"),
        scratch_types=[
            pltpu.VMEM((CHUNK,), jnp.float32),
            pltpu.VMEM((CHUNK,), jnp.float32),
            pltpu.VMEM((CHUNK,), jnp.float32),
            pltpu.VMEM((ROWS_PER_W * LANES,), jnp.float32),
            pltpu.SemaphoreType.DMA,
            pltpu.SemaphoreType.DMA,
            pltpu.SemaphoreType.DMA,
            pltpu.SemaphoreType.DMA,
            pltpu.SemaphoreType.DMA,
            pltpu.SemaphoreType.DMA,
        ],
    )
    out = k(x_flat, signs_rep)
    return out.reshape(x.shape)


# speedup vs baseline: 3.1583x; 1.0063x over previous
import jax
import jax.numpy as jnp
import numpy as np
from jax import lax
from jax.experimental import pallas as pl
from jax.experimental.pallas import tpu as pltpu, tpu_sc as plsc

P = 0.5
ROWS = 64
COLS = 160000
NC, NS = 2, 16
NW = NC * NS
CHUNK = 40000
CHUNKS_PER_ROW = COLS // CHUNK
KA = ROWS * CHUNKS_PER_ROW // NW
LANES = 16

_MASK = (1, 0, 0, 0, 1, 0, 1, 0, 0, 0, 1, 0, 0, 0, 1, 0,
         0, 0, 1, 0, 1, 1, 1, 0, 1, 0, 1, 1, 0, 0, 0, 1,
         1, 0, 0, 1, 0, 0, 1, 1, 1, 0, 1, 0, 0, 1, 0, 0,
         0, 1, 0, 1, 1, 0, 0, 1, 1, 1, 0, 0, 1, 1, 0, 1)


def _plan():
    neg = [(r * COLS + c * CHUNK) // 8
           for r in range(ROWS) if _MASK[r] for c in range(CHUNKS_PER_ROW)]
    cpy = [(r * COLS + c * CHUNK) // 8
           for r in range(ROWS) if not _MASK[r] for c in range(CHUNKS_PER_ROW)]
    oc = np.zeros((NW, 16), dtype=np.int32)
    fl = np.zeros((NW, 16), dtype=np.int32)
    per_w = [[] for _ in range(NW)]
    per_f = [[] for _ in range(NW)]
    for i, off in enumerate(neg):
        per_w[i % NW].append(off)
        per_f[i % NW].append(1)
    ptr = 0
    for w in range(NW):
        while len(per_w[w]) < KA:
            per_w[w].append(cpy[ptr])
            per_f[w].append(0)
            ptr += 1
    assert ptr == len(cpy)
    for w in range(NW):
        oc[w, :KA] = per_w[w]
        fl[w, :KA] = per_f[w]
    return oc, fl


def _sc_body(x_hbm, oc_hbm, fl_hbm, out_hbm, b0, b1, b2, ocv, flv,
             si0, si1, si2, so0, so1, so2):
    bufs = (b0, b1, b2)
    sin = (si0, si1, si2)
    sout = (so0, so1, so2)
    wid = lax.axis_index("s") * NC + lax.axis_index("c")
    pltpu.sync_copy(oc_hbm.at[wid], ocv)
    pltpu.sync_copy(fl_hbm.at[wid], flv)
    ocvec = ocv[...]
    flvec = flv[...]
    offs = [ocvec[j] * 8 for j in range(KA)]
    negs = [flvec[j] for j in range(KA)]

    def in_start(j):
        pltpu.async_copy(x_hbm.at[pl.ds(offs[j], CHUNK)],
                         bufs[j % 3], sin[j % 3])

    def in_wait(j):
        pltpu.make_async_copy(x_hbm.at[pl.ds(offs[j], CHUNK)],
                              bufs[j % 3], sin[j % 3]).wait()

    def compute(j):
        buf = bufs[j % 3]

        @pl.when(negs[j] != 0)
        def _():
            @plsc.parallel_loop(0, CHUNK, LANES, unroll=8)
            def _body(i):
                sl = pl.ds(i, LANES)
                buf[sl] = -buf[sl]

    def out_start(j):
        pltpu.async_copy(bufs[j % 3], out_hbm.at[pl.ds(offs[j], CHUNK)],
                         sout[j % 3])

    def out_wait(j):
        pltpu.make_async_copy(bufs[j % 3], out_hbm.at[pl.ds(offs[j], CHUNK)],
                              sout[j % 3]).wait()

    in_start(0)
    in_start(1)
    for j in range(KA):
        in_wait(j)
        compute(j)
        out_start(j)
        if j + 2 < KA:
            if j - 1 >= 0:
                out_wait(j - 1)
            in_start(j + 2)
    for j in range(max(KA - 3, 0), KA):
        out_wait(j)


_OC, _FL = _plan()


def kernel(x):
    x_flat = x.reshape(-1)
    k = pl.kernel(
        _sc_body,
        out_type=jax.ShapeDtypeStruct((ROWS * COLS,), jnp.float32),
        mesh=plsc.VectorSubcoreMesh(core_axis_name="c", subcore_axis_name="s"),
        scratch_types=[
            pltpu.VMEM((CHUNK,), jnp.float32),
            pltpu.VMEM((CHUNK,), jnp.float32),
            pltpu.VMEM((CHUNK,), jnp.float32),
            pltpu.VMEM((16,), jnp.int32),
            pltpu.VMEM((16,), jnp.int32),
            pltpu.SemaphoreType.DMA,
            pltpu.SemaphoreType.DMA,
            pltpu.SemaphoreType.DMA,
            pltpu.SemaphoreType.DMA,
            pltpu.SemaphoreType.DMA,
            pltpu.SemaphoreType.DMA,
        ],
    )
    out = k(x_flat, jnp.asarray(_OC), jnp.asarray(_FL))
    return out.reshape(x.shape)


# speedup vs baseline: 3.1599x; 1.0005x over previous
import jax
import jax.numpy as jnp
import numpy as np
from jax import lax
from jax.experimental import pallas as pl
from jax.experimental.pallas import tpu as pltpu, tpu_sc as plsc

P = 0.5
ROWS = 64
COLS = 160000
NC, NS = 2, 16
NW = NC * NS
CHUNK = 32000
CHUNKS_PER_ROW = COLS // CHUNK
KA = ROWS * CHUNKS_PER_ROW // NW
LANES = 16

_MASK = (1, 0, 0, 0, 1, 0, 1, 0, 0, 0, 1, 0, 0, 0, 1, 0,
         0, 0, 1, 0, 1, 1, 1, 0, 1, 0, 1, 1, 0, 0, 0, 1,
         1, 0, 0, 1, 0, 0, 1, 1, 1, 0, 1, 0, 0, 1, 0, 0,
         0, 1, 0, 1, 1, 0, 0, 1, 1, 1, 0, 0, 1, 1, 0, 1)


def _plan():
    neg = [(r * COLS + c * CHUNK) // 8
           for r in range(ROWS) if _MASK[r] for c in range(CHUNKS_PER_ROW)]
    cpy = [(r * COLS + c * CHUNK) // 8
           for r in range(ROWS) if not _MASK[r] for c in range(CHUNKS_PER_ROW)]
    oc = np.zeros((NW, 16), dtype=np.int32)
    fl = np.zeros((NW, 16), dtype=np.int32)
    per_w = [[] for _ in range(NW)]
    per_f = [[] for _ in range(NW)]
    for i, off in enumerate(neg):
        per_w[i % NW].append(off)
        per_f[i % NW].append(1)
    ptr = 0
    for w in range(NW):
        while len(per_w[w]) < KA:
            per_w[w].append(cpy[ptr])
            per_f[w].append(0)
            ptr += 1
    assert ptr == len(cpy)
    for w in range(NW):
        oc[w, :KA] = per_w[w]
        fl[w, :KA] = per_f[w]
    return oc, fl


def _sc_body(x_hbm, oc_hbm, fl_hbm, out_hbm, b0, b1, b2, b3, ocv, flv,
             si0, si1, si2, si3, so0, so1, so2, so3):
    bufs = (b0, b1, b2, b3)
    sin = (si0, si1, si2, si3)
    sout = (so0, so1, so2, so3)
    wid = lax.axis_index("s") * NC + lax.axis_index("c")
    pltpu.sync_copy(oc_hbm.at[wid], ocv)
    pltpu.sync_copy(fl_hbm.at[wid], flv)
    ocvec = ocv[...]
    flvec = flv[...]
    offs = [ocvec[j] * 8 for j in range(KA)]
    negs = [flvec[j] for j in range(KA)]

    def in_start(j):
        pltpu.async_copy(x_hbm.at[pl.ds(offs[j], CHUNK)],
                         bufs[j % 4], sin[j % 4])

    def in_wait(j):
        pltpu.make_async_copy(x_hbm.at[pl.ds(offs[j], CHUNK)],
                              bufs[j % 4], sin[j % 4]).wait()

    def compute(j):
        buf = bufs[j % 4]

        @pl.when(negs[j] != 0)
        def _():
            @plsc.parallel_loop(0, CHUNK, LANES, unroll=8)
            def _body(i):
                sl = pl.ds(i, LANES)
                buf[sl] = -buf[sl]

    def out_start(j):
        pltpu.async_copy(bufs[j % 4], out_hbm.at[pl.ds(offs[j], CHUNK)],
                         sout[j % 4])

    def out_wait(j):
        pltpu.make_async_copy(bufs[j % 4], out_hbm.at[pl.ds(offs[j], CHUNK)],
                              sout[j % 4]).wait()

    in_start(0)
    in_start(1)
    in_start(2)
    for j in range(KA):
        in_wait(j)
        compute(j)
        out_start(j)
        if j + 3 < KA:
            if j - 1 >= 0:
                out_wait(j - 1)
            in_start(j + 3)
    for j in range(max(KA - 4, 0), KA):
        out_wait(j)


_OC, _FL = _plan()


def kernel(x):
    x_flat = x.reshape(-1)
    k = pl.kernel(
        _sc_body,
        out_type=jax.ShapeDtypeStruct((ROWS * COLS,), jnp.float32),
        mesh=plsc.VectorSubcoreMesh(core_axis_name="c", subcore_axis_name="s"),
        scratch_types=[
            pltpu.VMEM((CHUNK,), jnp.float32),
            pltpu.VMEM((CHUNK,), jnp.float32),
            pltpu.VMEM((CHUNK,), jnp.float32),
            pltpu.VMEM((CHUNK,), jnp.float32),
            pltpu.VMEM((16,), jnp.int32),
            pltpu.VMEM((16,), jnp.int32),
            pltpu.SemaphoreType.DMA,
            pltpu.SemaphoreType.DMA,
            pltpu.SemaphoreType.DMA,
            pltpu.SemaphoreType.DMA,
            pltpu.SemaphoreType.DMA,
            pltpu.SemaphoreType.DMA,
            pltpu.SemaphoreType.DMA,
            pltpu.SemaphoreType.DMA,
        ],
    )
    out = k(x_flat, jnp.asarray(_OC), jnp.asarray(_FL))
    return out.reshape(x.shape)


# speedup vs baseline: 3.2299x; 1.0222x over previous
import jax
import jax.numpy as jnp
from jax import lax
from jax.experimental import pallas as pl
from jax.experimental.pallas import tpu as pltpu, tpu_sc as plsc

P = 0.5
ROWS = 64
COLS = 160000
NC, NS = 2, 16
NW = NC * NS
CHUNK = 40000
CHUNKS_PER_ROW = COLS // CHUNK
KA = 2 * CHUNKS_PER_ROW
LANES = 16

_MASK = (1, 0, 0, 0, 1, 0, 1, 0, 0, 0, 1, 0, 0, 0, 1, 0,
         0, 0, 1, 0, 1, 1, 1, 0, 1, 0, 1, 1, 0, 0, 0, 1,
         1, 0, 0, 1, 0, 0, 1, 1, 1, 0, 1, 0, 0, 1, 0, 0,
         0, 1, 0, 1, 1, 0, 0, 1, 1, 1, 0, 0, 1, 1, 0, 1)
def _bits32(bits):
    v = sum(b << i for i, b in enumerate(bits))
    return v - (1 << 32) if v >= (1 << 31) else v


_MLO = _bits32(_MASK[:32])
_MHI = _bits32(_MASK[32:])


def _sc_body(x_hbm, out_hbm, b0, b1, b2, si0, si1, si2, so0, so1, so2):
    bufs = (b0, b1, b2)
    sin = (si0, si1, si2)
    sout = (so0, so1, so2)
    wid = lax.axis_index("s") * NC + lax.axis_index("c")
    base = wid * (2 * COLS)
    mlo = jnp.int32(_MLO)
    mhi = jnp.int32(_MHI)

    def row_flag(p):
        row = 2 * wid + p
        lo_sh = jnp.minimum(row, 31)
        hi_sh = jnp.maximum(row - 32, 0)
        bits = jnp.where(row < 32,
                         lax.shift_right_logical(mlo, lo_sh),
                         lax.shift_right_logical(mhi, hi_sh))
        return (bits & 1) != 0

    negs = [row_flag(0), row_flag(1)]

    def off(j):
        return base + (j // CHUNKS_PER_ROW) * COLS + (j % CHUNKS_PER_ROW) * CHUNK

    def in_start(j):
        pltpu.async_copy(x_hbm.at[pl.ds(off(j), CHUNK)],
                         bufs[j % 3], sin[j % 3])

    def in_wait(j):
        pltpu.make_async_copy(x_hbm.at[pl.ds(off(j), CHUNK)],
                              bufs[j % 3], sin[j % 3]).wait()

    def compute(j):
        buf = bufs[j % 3]

        @pl.when(negs[j // CHUNKS_PER_ROW])
        def _():
            @plsc.parallel_loop(0, CHUNK, LANES, unroll=8)
            def _body(i):
                sl = pl.ds(i, LANES)
                buf[sl] = -buf[sl]

    def out_start(j):
        pltpu.async_copy(bufs[j % 3], out_hbm.at[pl.ds(off(j), CHUNK)],
                         sout[j % 3])

    def out_wait(j):
        pltpu.make_async_copy(bufs[j % 3], out_hbm.at[pl.ds(off(j), CHUNK)],
                              sout[j % 3]).wait()

    in_start(0)
    in_start(1)
    for j in range(KA):
        in_wait(j)
        compute(j)
        out_start(j)
        if j + 2 < KA:
            if j - 1 >= 0:
                out_wait(j - 1)
            in_start(j + 2)
    for j in range(max(KA - 3, 0), KA):
        out_wait(j)


def kernel(x):
    x_flat = x.reshape(-1)
    k = pl.kernel(
        _sc_body,
        out_type=jax.ShapeDtypeStruct((ROWS * COLS,), jnp.float32),
        mesh=plsc.VectorSubcoreMesh(core_axis_name="c", subcore_axis_name="s"),
        scratch_types=[
            pltpu.VMEM((CHUNK,), jnp.float32),
            pltpu.VMEM((CHUNK,), jnp.float32),
            pltpu.VMEM((CHUNK,), jnp.float32),
            pltpu.SemaphoreType.DMA,
            pltpu.SemaphoreType.DMA,
            pltpu.SemaphoreType.DMA,
            pltpu.SemaphoreType.DMA,
            pltpu.SemaphoreType.DMA,
            pltpu.SemaphoreType.DMA,
        ],
    )
    out = k(x_flat)
    return out.reshape(x.shape)


# speedup vs baseline: 3.2625x; 1.0101x over previous
import jax
import jax.numpy as jnp
from jax import lax
from jax.experimental import pallas as pl
from jax.experimental.pallas import tpu as pltpu, tpu_sc as plsc

P = 0.5
ROWS = 64
COLS = 160000
NC, NS = 2, 16
NW = NC * NS
CHUNK = 20000
CHUNKS_PER_ROW = COLS // CHUNK
KA = 2 * CHUNKS_PER_ROW
LANES = 16

_MASK = (1, 0, 0, 0, 1, 0, 1, 0, 0, 0, 1, 0, 0, 0, 1, 0,
         0, 0, 1, 0, 1, 1, 1, 0, 1, 0, 1, 1, 0, 0, 0, 1,
         1, 0, 0, 1, 0, 0, 1, 1, 1, 0, 1, 0, 0, 1, 0, 0,
         0, 1, 0, 1, 1, 0, 0, 1, 1, 1, 0, 0, 1, 1, 0, 1)
def _bits32(bits):
    v = sum(b << i for i, b in enumerate(bits))
    return v - (1 << 32) if v >= (1 << 31) else v


_MLO = _bits32(_MASK[:32])
_MHI = _bits32(_MASK[32:])


def _sc_body(x_hbm, out_hbm, b0, b1, b2, b3, b4, b5,
             si0, si1, si2, si3, si4, si5, so0, so1, so2, so3, so4, so5):
    bufs = (b0, b1, b2, b3, b4, b5)
    sin = (si0, si1, si2, si3, si4, si5)
    sout = (so0, so1, so2, so3, so4, so5)
    wid = lax.axis_index("s") * NC + lax.axis_index("c")
    base = wid * (2 * COLS)
    mlo = jnp.int32(_MLO)
    mhi = jnp.int32(_MHI)

    def row_flag(p):
        row = 2 * wid + p
        lo_sh = jnp.minimum(row, 31)
        hi_sh = jnp.maximum(row - 32, 0)
        bits = jnp.where(row < 32,
                         lax.shift_right_logical(mlo, lo_sh),
                         lax.shift_right_logical(mhi, hi_sh))
        return (bits & 1) != 0

    negs = [row_flag(0), row_flag(1)]

    def off(j):
        return base + (j // CHUNKS_PER_ROW) * COLS + (j % CHUNKS_PER_ROW) * CHUNK

    def in_start(j):
        pltpu.async_copy(x_hbm.at[pl.ds(off(j), CHUNK)],
                         bufs[j % 6], sin[j % 6])

    def in_wait(j):
        pltpu.make_async_copy(x_hbm.at[pl.ds(off(j), CHUNK)],
                              bufs[j % 6], sin[j % 6]).wait()

    def compute(j):
        buf = bufs[j % 6]

        @pl.when(negs[j // CHUNKS_PER_ROW])
        def _():
            @plsc.parallel_loop(0, CHUNK, LANES, unroll=8)
            def _body(i):
                sl = pl.ds(i, LANES)
                buf[sl] = -buf[sl]

    def out_start(j):
        pltpu.async_copy(bufs[j % 6], out_hbm.at[pl.ds(off(j), CHUNK)],
                         sout[j % 6])

    def out_wait(j):
        pltpu.make_async_copy(bufs[j % 6], out_hbm.at[pl.ds(off(j), CHUNK)],
                              sout[j % 6]).wait()

    for j in range(5):
        in_start(j)
    for j in range(KA):
        in_wait(j)
        compute(j)
        out_start(j)
        if j + 5 < KA:
            if j - 1 >= 0:
                out_wait(j - 1)
            in_start(j + 5)
    for j in range(max(KA - 6, 0), KA):
        out_wait(j)


def kernel(x):
    x_flat = x.reshape(-1)
    k = pl.kernel(
        _sc_body,
        out_type=jax.ShapeDtypeStruct((ROWS * COLS,), jnp.float32),
        mesh=plsc.VectorSubcoreMesh(core_axis_name="c", subcore_axis_name="s"),
        scratch_types=(
            [pltpu.VMEM((CHUNK,), jnp.float32)] * 6
            + [pltpu.SemaphoreType.DMA] * 12
        ),
    )
    out = k(x_flat)
    return out.reshape(x.shape)
